# baseline (speedup 1.0000x reference)
import functools

import jax
import jax.numpy as jnp
from jax import lax
from jax.experimental import pallas as pl
from jax.experimental.pallas import tpu as pltpu
from jax.experimental.pallas import tpu_sc as plsc

VOCAB = 1000000
HIDDEN = 64
SEQ_LEN = 200
BATCH = 4096
N = BATCH * SEQ_LEN
NC, NS, LANES = 2, 16, 16
NW = NC * NS
BBLK = BATCH // NW
HQ = HIDDEN // LANES
LN_EPS = 1e-8


def _rsqrt(x):
    i = lax.bitcast_convert_type(x, jnp.int32)
    i = jnp.int32(0x5F3759DF) - lax.shift_right_logical(i, 1)
    y = lax.bitcast_convert_type(i, jnp.float32)
    for _ in range(3):
        y = y * (1.5 - 0.5 * x * y * y)
    return y


def _emb_body(tok_hbm, word_hbm, pos_hbm, gam_hbm, bet_hbm, out_hbm,
              idx_v, idx2_v, rows_v, out_v, pos_v, gv, bv, sem):
    wid = lax.axis_index("s") * NC + lax.axis_index("c")
    b0 = wid * BBLK

    pltpu.sync_copy(pos_hbm, pos_v)
    pltpu.sync_copy(gam_hbm, gv)
    pltpu.sync_copy(bet_hbm, bv)

    gvec = [gv[pl.ds(i * LANES, LANES)] for i in range(HQ)]
    bvec = [bv[pl.ds(i * LANES, LANES)] for i in range(HQ)]
    lane = lax.iota(jnp.int32, LANES)
    lane_hi = lax.shift_right_logical(lane, 1 + 1 + 1)
    colbase = lax.shift_left(lane & 7, 7)
    rowq = [lane_hi + 2 * q for q in range(HQ)]

    def l_body(l, carry):
        pltpu.sync_copy(tok_hbm.at[l, pl.ds(b0, BBLK)],
                        idx_v.at[pl.ds(0, BBLK)])
        for j in range(BBLK // LANES):
            sl = pl.ds(j * LANES, LANES)
            idx2_v[sl] = lax.shift_right_logical(idx_v[sl], 1)
        pltpu.async_copy(word_hbm.at[idx2_v], rows_v, sem).wait()

        pq = [pos_v[pl.ds(l * HIDDEN + i * LANES, LANES)] for i in range(HQ)]

        @plsc.parallel_loop(0, BBLK, 1, unroll=4)
        def token_body(t):
            tok = idx_v[pl.ds(t, LANES)][0]
            half = (tok & 1) * HIDDEN
            x = [rows_v[t, pl.ds(half + i * LANES, LANES)] + pq[i]
                 for i in range(HQ)]
            s = (x[0] + x[1]) + (x[2] + x[3])
            ss = (x[0] * x[0] + x[1] * x[1]) + (x[2] * x[2] + x[3] * x[3])
            mean = jnp.sum(s) * (1.0 / HIDDEN)
            var = jnp.sum(ss) * (1.0 / HIDDEN) - mean * mean
            rs = _rsqrt(var + LN_EPS)
            msk = jnp.where(tok != 0, jnp.float32(1.0), jnp.float32(0.0))
            rsm = rs * msk
            zero = lane & 0
            tvec = zero + t
            for i in range(HQ):
                y = (x[i] - mean) * rsm * gvec[i] + msk * bvec[i]
                plsc.store_scatter(out_v, [rowq[i], zero, lane & 7, tvec], y)

        pltpu.sync_copy(out_v, out_hbm.at[l, :, pl.ds(wid, 1)])
        return carry

    lax.fori_loop(0, SEQ_LEN, l_body, 0)


_emb = functools.partial(
    pl.kernel,
    out_type=jax.ShapeDtypeStruct((SEQ_LEN, 8, NW, 8, BBLK), jnp.float32),
    mesh=plsc.VectorSubcoreMesh(core_axis_name="c", subcore_axis_name="s",
                                num_cores=NC, num_subcores=NS),
    compiler_params=pltpu.CompilerParams(needs_layout_passes=False,
                                         use_tc_tiling_on_sc=True),
    scratch_types=[
        pltpu.VMEM((BBLK + LANES,), jnp.int32),
        pltpu.VMEM((BBLK,), jnp.int32),
        pltpu.VMEM((BBLK, 2 * HIDDEN), jnp.float32),
        pltpu.VMEM((8, 1, 8, BBLK), jnp.float32),
        pltpu.VMEM((SEQ_LEN * HIDDEN,), jnp.float32),
        pltpu.VMEM((HIDDEN,), jnp.float32),
        pltpu.VMEM((HIDDEN,), jnp.float32),
        pltpu.SemaphoreType.DMA,
    ],
)(_emb_body)


@jax.jit
def kernel(tokens, word_table, pos_table, gamma, beta):
    tok_t = tokens.T.astype(jnp.int32)
    word2 = word_table.reshape(VOCAB // 2, 2 * HIDDEN)
    out = _emb(tok_t, word2, pos_table.reshape(-1), gamma, beta)
    z = out.transpose(2, 4, 0, 1, 3)
    return z.reshape(BATCH, SEQ_LEN, HIDDEN)


# speedup vs baseline: 1.1474x; 1.1474x over previous
import functools

import jax
import jax.numpy as jnp
from jax import lax
from jax.experimental import pallas as pl
from jax.experimental.pallas import tpu as pltpu
from jax.experimental.pallas import tpu_sc as plsc

VOCAB = 1000000
HIDDEN = 64
SEQ_LEN = 200
BATCH = 4096
N = BATCH * SEQ_LEN
NC, NS, LANES = 2, 16, 16
NW = NC * NS
BBLK = BATCH // NW
HQ = HIDDEN // LANES
CL = 2
CTOK = CL * BBLK
NCHUNK = SEQ_LEN // CL
LN_EPS = 1e-8


def _rsqrt(x):
    i = lax.bitcast_convert_type(x, jnp.int32)
    i = jnp.int32(0x5F3759DF) - lax.shift_right_logical(i, 1)
    y = lax.bitcast_convert_type(i, jnp.float32)
    for _ in range(3):
        y = y * (1.5 - 0.5 * x * y * y)
    return y


def _emb_body(tok_hbm, word_hbm, pos_hbm, gam_hbm, bet_hbm, out_hbm,
              idxa, idxb, idx2a, idx2b, rowsa, rowsb, outa, outb,
              pos_v, gv, bv, sga, sgb, swa, swb):
    wid = lax.axis_index("s") * NC + lax.axis_index("c")
    b0 = wid * BBLK

    pltpu.sync_copy(pos_hbm, pos_v)
    pltpu.sync_copy(gam_hbm, gv)
    pltpu.sync_copy(bet_hbm, bv)

    gvec = [gv[pl.ds(i * LANES, LANES)] for i in range(HQ)]
    bvec = [bv[pl.ds(i * LANES, LANES)] for i in range(HQ)]
    lane = lax.iota(jnp.int32, LANES)
    lane_hi = lax.shift_right_logical(lane, 3)
    lane_lo = lane & 7
    rowq = [lane_hi + 2 * q for q in range(HQ)]
    zero = lane & 0

    def issue_gather(c, idxf, idx2, rows, sem):
        l0 = c * CL
        for li in range(CL):
            pltpu.sync_copy(tok_hbm.at[l0 + li, pl.ds(b0, BBLK)],
                            idxf.at[pl.ds(li * BBLK, BBLK)])
        for li in range(CL):
            for j in range(BBLK // LANES):
                sl = pl.ds(li * BBLK + j * LANES, LANES)
                idx2[li, pl.ds(j * LANES, LANES)] = \
                    lax.shift_right_logical(idxf[sl], 1)
        for li in range(CL):
            pltpu.async_copy(word_hbm.at[idx2.at[li]],
                             rows.at[pl.ds(li * BBLK, BBLK)], sem)

    def wait_gather(idx2, rows, sem):
        for li in range(CL):
            pltpu.make_async_copy(word_hbm.at[idx2.at[li]],
                                  rows.at[pl.ds(li * BBLK, BBLK)], sem).wait()

    def compute(c, idxf, rows, outv):
        l0 = c * CL
        for li in range(CL):
            pq = [pos_v[pl.ds((l0 + li) * HIDDEN + i * LANES, LANES)]
                  for i in range(HQ)]
            livec = zero + li

            @plsc.parallel_loop(0, BBLK, 1, unroll=4)
            def token_body(t):
                g = li * BBLK + t
                tok = idxf[pl.ds(g, LANES)][0]
                half = (tok & 1) * HIDDEN
                x = [rows[g, pl.ds(half + i * LANES, LANES)] + pq[i]
                     for i in range(HQ)]
                s = (x[0] + x[1]) + (x[2] + x[3])
                ss = (x[0] * x[0] + x[1] * x[1]) + (x[2] * x[2] + x[3] * x[3])
                mean = jnp.sum(s) * (1.0 / HIDDEN)
                var = jnp.sum(ss) * (1.0 / HIDDEN) - mean * mean
                rs = _rsqrt(var + LN_EPS)
                msk = jnp.where(tok != 0, jnp.float32(1.0), jnp.float32(0.0))
                rsm = rs * msk
                tvec = zero + t
                for i in range(HQ):
                    y = (x[i] - mean) * rsm * gvec[i] + msk * bvec[i]
                    plsc.store_scatter(outv,
                                       [livec, rowq[i], zero, lane_lo, tvec],
                                       y)

    def issue_wb(c, outv, sem):
        pltpu.async_copy(outv, out_hbm.at[pl.ds(c * CL, CL), :,
                                          pl.ds(wid, 1)], sem)

    def wait_wb(outv, sem):
        pltpu.make_async_copy(outv, out_hbm.at[pl.ds(0, CL), :,
                                               pl.ds(wid, 1)], sem).wait()

    issue_gather(0, idxa, idx2a, rowsa, sga)
    issue_gather(1, idxb, idx2b, rowsb, sgb)
    wait_gather(idx2a, rowsa, sga)
    compute(0, idxa, rowsa, outa)
    issue_wb(0, outa, swa)
    issue_gather(2, idxa, idx2a, rowsa, sga)
    wait_gather(idx2b, rowsb, sgb)
    compute(1, idxb, rowsb, outb)
    issue_wb(1, outb, swb)
    issue_gather(3, idxb, idx2b, rowsb, sgb)

    def body(k, carry):
        c = 2 * k
        wait_gather(idx2a, rowsa, sga)
        wait_wb(outa, swa)
        compute(c, idxa, rowsa, outa)
        issue_wb(c, outa, swa)
        issue_gather(c + 2, idxa, idx2a, rowsa, sga)
        wait_gather(idx2b, rowsb, sgb)
        wait_wb(outb, swb)
        compute(c + 1, idxb, rowsb, outb)
        issue_wb(c + 1, outb, swb)
        issue_gather(c + 3, idxb, idx2b, rowsb, sgb)
        return carry

    lax.fori_loop(1, NCHUNK // 2 - 1, body, 0)

    wait_gather(idx2a, rowsa, sga)
    wait_wb(outa, swa)
    compute(NCHUNK - 2, idxa, rowsa, outa)
    issue_wb(NCHUNK - 2, outa, swa)
    wait_gather(idx2b, rowsb, sgb)
    wait_wb(outb, swb)
    compute(NCHUNK - 1, idxb, rowsb, outb)
    issue_wb(NCHUNK - 1, outb, swb)
    wait_wb(outa, swa)
    wait_wb(outb, swb)


_emb = functools.partial(
    pl.kernel,
    out_type=jax.ShapeDtypeStruct((SEQ_LEN, 8, NW, 8, BBLK), jnp.float32),
    mesh=plsc.VectorSubcoreMesh(core_axis_name="c", subcore_axis_name="s",
                                num_cores=NC, num_subcores=NS),
    compiler_params=pltpu.CompilerParams(needs_layout_passes=False,
                                         use_tc_tiling_on_sc=True),
    scratch_types=[
        pltpu.VMEM((CTOK + LANES,), jnp.int32),
        pltpu.VMEM((CTOK + LANES,), jnp.int32),
        pltpu.VMEM((CL, BBLK), jnp.int32),
        pltpu.VMEM((CL, BBLK), jnp.int32),
        pltpu.VMEM((CTOK, 2 * HIDDEN), jnp.float32),
        pltpu.VMEM((CTOK, 2 * HIDDEN), jnp.float32),
        pltpu.VMEM((CL, 8, 1, 8, BBLK), jnp.float32),
        pltpu.VMEM((CL, 8, 1, 8, BBLK), jnp.float32),
        pltpu.VMEM((SEQ_LEN * HIDDEN,), jnp.float32),
        pltpu.VMEM((HIDDEN,), jnp.float32),
        pltpu.VMEM((HIDDEN,), jnp.float32),
        pltpu.SemaphoreType.DMA,
        pltpu.SemaphoreType.DMA,
        pltpu.SemaphoreType.DMA,
        pltpu.SemaphoreType.DMA,
    ],
)(_emb_body)


@jax.jit
def kernel(tokens, word_table, pos_table, gamma, beta):
    tok_t = tokens.T.astype(jnp.int32)
    word2 = word_table.reshape(VOCAB // 2, 2 * HIDDEN)
    out = _emb(tok_t, word2, pos_table.reshape(-1), gamma, beta)
    z = out.transpose(2, 4, 0, 1, 3)
    return z.reshape(BATCH, SEQ_LEN, HIDDEN)


# speedup vs baseline: 1.2154x; 1.0593x over previous
import functools

import jax
import jax.numpy as jnp
from jax import lax
from jax.experimental import pallas as pl
from jax.experimental.pallas import tpu as pltpu
from jax.experimental.pallas import tpu_sc as plsc

VOCAB = 1000000
HIDDEN = 64
SEQ_LEN = 200
BATCH = 4096
N = BATCH * SEQ_LEN
NC, NS, LANES = 2, 16, 16
NW = NC * NS
BBLK = BATCH // NW
HQ = HIDDEN // LANES
CL = 2
CTOK = CL * BBLK
NCHUNK = SEQ_LEN // CL
LN_EPS = 1e-8


def _rsqrt(x):
    i = lax.bitcast_convert_type(x, jnp.int32)
    i = jnp.int32(0x5F3759DF) - lax.shift_right_logical(i, 1)
    y = lax.bitcast_convert_type(i, jnp.float32)
    for _ in range(3):
        y = y * (1.5 - 0.5 * x * y * y)
    return y


def _emb_body(tok_hbm, word_hbm, pos_hbm, gam_hbm, bet_hbm, out_hbm,
              idxa, idxb, rowsa, rowsb, outa, outb,
              pos_v, gv, bv, sga, sgb, swa, swb):
    wid = lax.axis_index("s") * NC + lax.axis_index("c")
    b0 = wid * BBLK

    pltpu.sync_copy(pos_hbm, pos_v)
    pltpu.sync_copy(gam_hbm, gv)
    pltpu.sync_copy(bet_hbm, bv)

    gvec = [gv[pl.ds(i * LANES, LANES)] for i in range(HQ)]
    bvec = [bv[pl.ds(i * LANES, LANES)] for i in range(HQ)]
    lane = lax.iota(jnp.int32, LANES)
    lane_hi = lax.shift_right_logical(lane, 3)
    lane_lo = lane & 7
    rowq = [lane_hi + 2 * q for q in range(HQ)]
    zero = lane & 0

    def issue_gather(c, idxf, rows, sem):
        l0 = c * CL
        for li in range(CL):
            pltpu.sync_copy(tok_hbm.at[l0 + li, pl.ds(b0, BBLK)],
                            idxf.at[pl.ds(li * BBLK, BBLK)])
        for li in range(CL):
            pltpu.async_copy(word_hbm.at[idxf.at[pl.ds(li * BBLK, BBLK)]],
                             rows.at[pl.ds(li * BBLK, BBLK)], sem)

    def wait_gather(idxf, rows, sem):
        for li in range(CL):
            pltpu.make_async_copy(word_hbm.at[idxf.at[pl.ds(li * BBLK, BBLK)]],
                                  rows.at[pl.ds(li * BBLK, BBLK)], sem).wait()

    def compute(c, idxf, rows, outv):
        l0 = c * CL
        for li in range(CL):
            pq = [pos_v[pl.ds((l0 + li) * HIDDEN + i * LANES, LANES)]
                  for i in range(HQ)]
            livec = zero + li

            @plsc.parallel_loop(0, BBLK, 1, unroll=4)
            def token_body(t):
                g = li * BBLK + t
                tok = idxf[pl.ds(g, LANES)][0]
                x = [rows[g, pl.ds(i * LANES, LANES)] + pq[i]
                     for i in range(HQ)]
                s = (x[0] + x[1]) + (x[2] + x[3])
                ss = (x[0] * x[0] + x[1] * x[1]) + (x[2] * x[2] + x[3] * x[3])
                mean = jnp.sum(s) * (1.0 / HIDDEN)
                var = jnp.sum(ss) * (1.0 / HIDDEN) - mean * mean
                rs = _rsqrt(var + LN_EPS)
                msk = jnp.where(tok != 0, jnp.float32(1.0), jnp.float32(0.0))
                rsm = rs * msk
                tvec = zero + t
                for i in range(HQ):
                    y = (x[i] - mean) * rsm * gvec[i] + msk * bvec[i]
                    plsc.store_scatter(outv,
                                       [livec, rowq[i], zero, lane_lo, tvec],
                                       y)

    def issue_wb(c, outv, sem):
        pltpu.async_copy(outv, out_hbm.at[pl.ds(c * CL, CL), :,
                                          pl.ds(wid, 1)], sem)

    def wait_wb(outv, sem):
        pltpu.make_async_copy(outv, out_hbm.at[pl.ds(0, CL), :,
                                               pl.ds(wid, 1)], sem).wait()

    issue_gather(0, idxa, rowsa, sga)
    issue_gather(1, idxb, rowsb, sgb)
    wait_gather(idxa, rowsa, sga)
    compute(0, idxa, rowsa, outa)
    issue_wb(0, outa, swa)
    issue_gather(2, idxa, rowsa, sga)
    wait_gather(idxb, rowsb, sgb)
    compute(1, idxb, rowsb, outb)
    issue_wb(1, outb, swb)
    issue_gather(3, idxb, rowsb, sgb)

    def body(k, carry):
        c = 2 * k
        wait_gather(idxa, rowsa, sga)
        wait_wb(outa, swa)
        compute(c, idxa, rowsa, outa)
        issue_wb(c, outa, swa)
        issue_gather(c + 2, idxa, rowsa, sga)
        wait_gather(idxb, rowsb, sgb)
        wait_wb(outb, swb)
        compute(c + 1, idxb, rowsb, outb)
        issue_wb(c + 1, outb, swb)
        issue_gather(c + 3, idxb, rowsb, sgb)
        return carry

    lax.fori_loop(1, NCHUNK // 2 - 1, body, 0)

    wait_gather(idxa, rowsa, sga)
    wait_wb(outa, swa)
    compute(NCHUNK - 2, idxa, rowsa, outa)
    issue_wb(NCHUNK - 2, outa, swa)
    wait_gather(idxb, rowsb, sgb)
    wait_wb(outb, swb)
    compute(NCHUNK - 1, idxb, rowsb, outb)
    issue_wb(NCHUNK - 1, outb, swb)
    wait_wb(outa, swa)
    wait_wb(outb, swb)


_emb = functools.partial(
    pl.kernel,
    out_type=jax.ShapeDtypeStruct((SEQ_LEN, 8, NW, 8, BBLK), jnp.float32),
    mesh=plsc.VectorSubcoreMesh(core_axis_name="c", subcore_axis_name="s",
                                num_cores=NC, num_subcores=NS),
    compiler_params=pltpu.CompilerParams(needs_layout_passes=False,
                                         use_tc_tiling_on_sc=False),
    scratch_types=[
        pltpu.VMEM((CTOK + LANES,), jnp.int32),
        pltpu.VMEM((CTOK + LANES,), jnp.int32),
        pltpu.VMEM((CTOK, HIDDEN), jnp.float32),
        pltpu.VMEM((CTOK, HIDDEN), jnp.float32),
        pltpu.VMEM((CL, 8, 1, 8, BBLK), jnp.float32),
        pltpu.VMEM((CL, 8, 1, 8, BBLK), jnp.float32),
        pltpu.VMEM((SEQ_LEN * HIDDEN,), jnp.float32),
        pltpu.VMEM((HIDDEN,), jnp.float32),
        pltpu.VMEM((HIDDEN,), jnp.float32),
        pltpu.SemaphoreType.DMA,
        pltpu.SemaphoreType.DMA,
        pltpu.SemaphoreType.DMA,
        pltpu.SemaphoreType.DMA,
    ],
)(_emb_body)


@jax.jit
def kernel(tokens, word_table, pos_table, gamma, beta):
    tok_t = tokens.T.astype(jnp.int32)
    out = _emb(tok_t, word_table, pos_table.reshape(-1), gamma, beta)
    z = out.transpose(2, 4, 0, 1, 3)
    return z.reshape(BATCH, SEQ_LEN, HIDDEN)
